# Initial kernel scaffold; baseline (speedup 1.0000x reference)
import jax
import jax.numpy as jnp
from jax.experimental import pallas as pl


def kernel(inputs):
    raise NotImplementedError("write your pallas kernel here")


# speedup vs baseline: 15.1581x; 15.1581x over previous
import functools

import jax
import jax.numpy as jnp
from jax import lax
from jax.experimental import pallas as pl

K_TOP_N = 8
SEQ = 8192
CH = 64


def _kmax_body(x_ref, out_ref):
    data = x_ref[0]
    iota = lax.broadcasted_iota(jnp.int32, (SEQ, CH), 0)
    for k in range(K_TOP_N):
        m = jnp.max(data, axis=0)
        hit = data == m[None, :]
        first = jnp.min(jnp.where(hit, iota, SEQ), axis=0)
        data = jnp.where(iota == first[None, :], -jnp.inf, data)
        out_ref[0, :, k] = m


def kernel(inputs):
    x = inputs
    b = x.shape[0]
    out3 = pl.pallas_call(
        _kmax_body,
        grid=(b,),
        in_specs=[pl.BlockSpec((1, SEQ, CH), lambda i: (i, 0, 0))],
        out_specs=pl.BlockSpec((1, CH, K_TOP_N), lambda i: (i, 0, 0)),
        out_shape=jax.ShapeDtypeStruct((b, CH, K_TOP_N), jnp.float32),
    )(x)
    return out3.reshape(b, CH * K_TOP_N)


# speedup vs baseline: 26.0081x; 1.7158x over previous
import jax
import jax.numpy as jnp
from jax import lax
from jax.experimental import pallas as pl
from jax.experimental.pallas import tpu as pltpu
from jax.experimental.pallas import tpu_sc as plsc

K_TOP_N = 8
SEQ = 8192
CH = 64
NC, NS, LANES = 2, 16, 16
NW = NC * NS
CGS = CH // LANES
CHUNK = 512
NCHUNK = SEQ // CHUNK
GRP = 8
NGRP = CHUNK // GRP
OUT_W = CH * K_TOP_N


def _sc_body(x_hbm, out_hbm, buf, stage):
    wid = lax.axis_index("s") * NC + lax.axis_index("c")
    bpw = x_hbm.shape[0] // NW
    neg_inf = jnp.full((LANES,), -jnp.inf, jnp.float32)

    def batch_body(i, _carry):
        b = wid * bpw + i

        def chunk_body(ci, T32):
            pltpu.sync_copy(x_hbm.at[b, pl.ds(ci * CHUNK, CHUNK), :], buf)
            Ts = list(T32)
            for cg in range(CGS):
                c0 = cg * LANES

                def grp_body(g, T):
                    s = g * GRP
                    v = [buf[s + j, pl.ds(c0, LANES)] for j in range(GRP)]
                    m = v[0]
                    for j in range(1, GRP):
                        m = jnp.maximum(m, v[j])
                    hit = jnp.any(m > T[K_TOP_N - 1])

                    def do_insert(T):
                        Tl = list(T)
                        for j in range(GRP):
                            new = v[j]
                            for k in range(K_TOP_N):
                                hi = jnp.maximum(Tl[k], new)
                                new = jnp.minimum(Tl[k], new)
                                Tl[k] = hi
                        return tuple(Tl)

                    return lax.cond(hit, do_insert, lambda T: T, T)

                Tcg = lax.fori_loop(
                    0, NGRP, grp_body,
                    tuple(Ts[cg * K_TOP_N:(cg + 1) * K_TOP_N]))
                Ts[cg * K_TOP_N:(cg + 1) * K_TOP_N] = list(Tcg)
            return tuple(Ts)

        T32 = lax.fori_loop(0, NCHUNK, chunk_body,
                            (neg_inf,) * (CGS * K_TOP_N))
        lane = lax.iota(jnp.int32, LANES)
        for cg in range(CGS):
            for j in range(K_TOP_N):
                idx = lane * K_TOP_N + (cg * LANES * K_TOP_N + j)
                plsc.store_scatter(stage, [idx], T32[cg * K_TOP_N + j])
        pltpu.sync_copy(stage, out_hbm.at[pl.ds(b * OUT_W, OUT_W)])
        return 0

    lax.fori_loop(0, bpw, batch_body, 0)


def kernel(inputs):
    x = inputs
    b = x.shape[0]
    mesh = plsc.VectorSubcoreMesh(core_axis_name="c", subcore_axis_name="s")
    f = pl.kernel(
        _sc_body,
        out_type=jax.ShapeDtypeStruct((b * OUT_W,), jnp.float32),
        mesh=mesh,
        scratch_types=[
            pltpu.VMEM((CHUNK, CH), jnp.float32),
            pltpu.VMEM((OUT_W,), jnp.float32),
        ],
        compiler_params=pltpu.CompilerParams(needs_layout_passes=False),
    )
    return f(x).reshape(b, OUT_W)


# speedup vs baseline: 30.8672x; 1.1868x over previous
import jax
import jax.numpy as jnp
from jax import lax
from jax.experimental import pallas as pl
from jax.experimental.pallas import tpu as pltpu
from jax.experimental.pallas import tpu_sc as plsc

K_TOP_N = 8
SEQ = 8192
CH = 64
NC, NS, LANES = 2, 16, 16
NW = NC * NS
CGS = CH // LANES
CHUNK = 256
NCHUNK = SEQ // CHUNK
GRP = 8
NGRP = CHUNK // GRP
OUT_W = CH * K_TOP_N


def _sc_body(x_hbm, out_hbm, buf0, buf1, stage, sem0, sem1):
    wid = lax.axis_index("s") * NC + lax.axis_index("c")
    bpw = x_hbm.shape[0] // NW
    neg_inf = jnp.full((LANES,), -jnp.inf, jnp.float32)

    def process(buf, T32):
        Ts = list(T32)
        for cg in range(CGS):
            c0 = cg * LANES

            def grp_body(g, T):
                s = g * GRP
                v = [buf[s + j, pl.ds(c0, LANES)] for j in range(GRP)]
                m = v[0]
                for j in range(1, GRP):
                    m = jnp.maximum(m, v[j])
                hit = jnp.any(m > T[K_TOP_N - 1])

                def do_insert(T):
                    Tl = list(T)
                    for j in range(GRP):
                        new = v[j]
                        for k in range(K_TOP_N):
                            hi = jnp.maximum(Tl[k], new)
                            new = jnp.minimum(Tl[k], new)
                            Tl[k] = hi
                    return tuple(Tl)

                return lax.cond(hit, do_insert, lambda T: T, T)

            Tcg = lax.fori_loop(
                0, NGRP, grp_body,
                tuple(Ts[cg * K_TOP_N:(cg + 1) * K_TOP_N]))
            Ts[cg * K_TOP_N:(cg + 1) * K_TOP_N] = list(Tcg)
        return tuple(Ts)

    def chunk_src(b, ci):
        return x_hbm.at[b, pl.ds(ci * CHUNK, CHUNK), :]

    def batch_body(i, _carry):
        b = wid * bpw + i

        pltpu.make_async_copy(chunk_src(b, 0), buf0, sem0).start()

        def pair_body(pi, T32):
            ci = pi * 2
            pltpu.make_async_copy(chunk_src(b, ci), buf0, sem0).wait()
            pltpu.make_async_copy(chunk_src(b, ci + 1), buf1, sem1).start()
            T32 = process(buf0, T32)

            @pl.when(pi < NCHUNK // 2 - 1)
            def _():
                pltpu.make_async_copy(chunk_src(b, ci + 2), buf0, sem0).start()

            pltpu.make_async_copy(chunk_src(b, ci + 1), buf1, sem1).wait()
            T32 = process(buf1, T32)
            return T32

        T32 = lax.fori_loop(0, NCHUNK // 2, pair_body,
                            (neg_inf,) * (CGS * K_TOP_N))
        lane = lax.iota(jnp.int32, LANES)
        for cg in range(CGS):
            for j in range(K_TOP_N):
                idx = lane * K_TOP_N + (cg * LANES * K_TOP_N + j)
                plsc.store_scatter(stage, [idx], T32[cg * K_TOP_N + j])
        pltpu.sync_copy(stage, out_hbm.at[pl.ds(b * OUT_W, OUT_W)])
        return 0

    lax.fori_loop(0, bpw, batch_body, 0)


def kernel(inputs):
    x = inputs
    b = x.shape[0]
    mesh = plsc.VectorSubcoreMesh(core_axis_name="c", subcore_axis_name="s")
    f = pl.kernel(
        _sc_body,
        out_type=jax.ShapeDtypeStruct((b * OUT_W,), jnp.float32),
        mesh=mesh,
        scratch_types=[
            pltpu.VMEM((CHUNK, CH), jnp.float32),
            pltpu.VMEM((CHUNK, CH), jnp.float32),
            pltpu.VMEM((OUT_W,), jnp.float32),
            pltpu.SemaphoreType.DMA,
            pltpu.SemaphoreType.DMA,
        ],
        compiler_params=pltpu.CompilerParams(needs_layout_passes=False),
    )
    return f(x).reshape(b, OUT_W)


# speedup vs baseline: 49.6920x; 1.6099x over previous
import jax
import jax.numpy as jnp
from jax import lax
from jax.experimental import pallas as pl
from jax.experimental.pallas import tpu as pltpu
from jax.experimental.pallas import tpu_sc as plsc

K_TOP_N = 8
SEQ = 8192
CH = 64
NC, NS, LANES = 2, 16, 16
NW = NC * NS
CGS = CH // LANES
CHUNK = 256
NCHUNK = SEQ // CHUNK
GRP = 8
NGRP = CHUNK // GRP
OUT_W = CH * K_TOP_N


SORT8 = ((0, 1), (2, 3), (4, 5), (6, 7),
         (0, 2), (1, 3), (4, 6), (5, 7),
         (1, 2), (5, 6),
         (0, 4), (1, 5), (2, 6), (3, 7),
         (2, 4), (3, 5),
         (1, 2), (3, 4), (5, 6))
BITONIC8 = ((0, 4), (1, 5), (2, 6), (3, 7),
            (0, 2), (1, 3), (4, 6), (5, 7),
            (0, 1), (2, 3), (4, 5), (6, 7))


def _sc_body(x_hbm, out_hbm, buf0, buf1, stage, sem0, sem1):
    wid = lax.axis_index("s") * NC + lax.axis_index("c")
    bpw = x_hbm.shape[0] // NW
    neg_inf = jnp.full((LANES,), -jnp.inf, jnp.float32)

    def process(buf, T32):
        def grp_body(g, Tflat):
            s = g * GRP
            out = []
            for cg in range(CGS):
                T = list(Tflat[cg * K_TOP_N:(cg + 1) * K_TOP_N])
                v = [buf[s + j, pl.ds(cg * LANES, LANES)] for j in range(GRP)]
                for (i, j) in SORT8:
                    hi = jnp.maximum(v[i], v[j])
                    lo = jnp.minimum(v[i], v[j])
                    v[i], v[j] = hi, lo
                m = [jnp.maximum(T[i], v[K_TOP_N - 1 - i])
                     for i in range(K_TOP_N)]
                for (i, j) in BITONIC8:
                    hi = jnp.maximum(m[i], m[j])
                    lo = jnp.minimum(m[i], m[j])
                    m[i], m[j] = hi, lo
                out += m
            return tuple(out)

        return lax.fori_loop(0, NGRP, grp_body, tuple(T32))

    def chunk_src(b, ci):
        return x_hbm.at[b, pl.ds(ci * CHUNK, CHUNK), :]

    def batch_body(i, _carry):
        b = wid * bpw + i

        pltpu.make_async_copy(chunk_src(b, 0), buf0, sem0).start()

        def pair_body(pi, T32):
            ci = pi * 2
            pltpu.make_async_copy(chunk_src(b, ci), buf0, sem0).wait()
            pltpu.make_async_copy(chunk_src(b, ci + 1), buf1, sem1).start()
            T32 = process(buf0, T32)

            @pl.when(pi < NCHUNK // 2 - 1)
            def _():
                pltpu.make_async_copy(chunk_src(b, ci + 2), buf0, sem0).start()

            pltpu.make_async_copy(chunk_src(b, ci + 1), buf1, sem1).wait()
            T32 = process(buf1, T32)
            return T32

        T32 = lax.fori_loop(0, NCHUNK // 2, pair_body,
                            (neg_inf,) * (CGS * K_TOP_N))
        lane = lax.iota(jnp.int32, LANES)
        for cg in range(CGS):
            for j in range(K_TOP_N):
                idx = lane * K_TOP_N + (cg * LANES * K_TOP_N + j)
                plsc.store_scatter(stage, [idx], T32[cg * K_TOP_N + j])
        pltpu.sync_copy(stage, out_hbm.at[pl.ds(b * OUT_W, OUT_W)])
        return 0

    lax.fori_loop(0, bpw, batch_body, 0)


def kernel(inputs):
    x = inputs
    b = x.shape[0]
    mesh = plsc.VectorSubcoreMesh(core_axis_name="c", subcore_axis_name="s")
    f = pl.kernel(
        _sc_body,
        out_type=jax.ShapeDtypeStruct((b * OUT_W,), jnp.float32),
        mesh=mesh,
        scratch_types=[
            pltpu.VMEM((CHUNK, CH), jnp.float32),
            pltpu.VMEM((CHUNK, CH), jnp.float32),
            pltpu.VMEM((OUT_W,), jnp.float32),
            pltpu.SemaphoreType.DMA,
            pltpu.SemaphoreType.DMA,
        ],
        compiler_params=pltpu.CompilerParams(needs_layout_passes=False),
    )
    return f(x).reshape(b, OUT_W)
